# Initial kernel scaffold; baseline (speedup 1.0000x reference)
import jax
import jax.numpy as jnp
from jax.experimental import pallas as pl


def kernel(h, x, edge_weight, weights, inp_weights, graph_mixing_weight, inp_graph_mixing_weight, edge_index):
    raise NotImplementedError("write your pallas kernel here")


# speedup vs baseline: 10.0359x; 10.0359x over previous
import functools

import jax
import jax.numpy as jnp
from jax import lax
from jax.experimental import pallas as pl
from jax.experimental.pallas import tpu as pltpu
from jax.experimental.pallas import tpu_sc as plsc


def _matmul_body(hx_ref, w_ref, o_ref):
    o_ref[...] = jnp.dot(hx_ref[...], w_ref[...],
                         preferred_element_type=jnp.float32,
                         precision=jax.lax.Precision.HIGHEST)


def _addrelu_body(a_ref, b_ref, o_ref):
    o_ref[...] = jnp.maximum(a_ref[...] + b_ref[...], 0.0)


def _make_sc_spmm(N, E, GK, D, DP):
    info = plsc.get_sparse_core_info()
    NC, NS, L = info.num_cores, info.num_subcores, info.num_lanes
    NW = NC * NS
    EW = E // NW
    B = 40
    T = EW // B
    WB = 40
    SB = ((N + NS - 1) // NS + WB - 1) // WB * WB
    NBLK = SB // WB
    assert E % NW == 0 and EW % B == 0 and T % 2 == 0 and N % WB == 0
    mesh = plsc.VectorSubcoreMesh(core_axis_name="c", subcore_axis_name="s")

    @functools.partial(
        pl.kernel,
        out_type=jax.ShapeDtypeStruct((NC, N, D), jnp.float32),
        mesh=mesh,
        compiler_params=pltpu.CompilerParams(needs_layout_passes=False),
        scratch_types=[
            pltpu.VMEM((2, B), jnp.int32),
            pltpu.VMEM((2, B), jnp.int32),
            pltpu.VMEM((GK * B,), jnp.float32),
            pltpu.VMEM((GK * B,), jnp.float32),
            pltpu.VMEM((B, DP), jnp.float32),
            pltpu.VMEM((B, DP), jnp.float32),
            pltpu.VMEM((B, D), jnp.float32),
            pltpu.VMEM_SHARED((N, D), jnp.float32),
            pltpu.SemaphoreType.DMA,
            pltpu.SemaphoreType.DMA,
        ],
    )
    def sc_spmm(p_hbm, ei_hbm, w_hbm, out_hbm,
                ei0_v, ei1_v, w0_v, w1_v, rows0_v, rows1_v, msg_v, acc_sh,
                sem0, sem1):
        cid = lax.axis_index("c")
        sid = lax.axis_index("s")
        wid = sid * NC + cid
        ei_v = (ei0_v, ei1_v)
        w_v = (w0_v, w1_v)
        rows_v = (rows0_v, rows1_v)
        sem = (sem0, sem1)

        def zero_row(r, carry):
            for dd in range(D // L):
                msg_v[r, pl.ds(dd * L, L)] = jnp.zeros((L,), jnp.float32)
            return carry
        lax.fori_loop(0, WB, zero_row, 0)
        row0 = sid * SB
        for b in range(NBLK):
            @pl.when(row0 + b * WB < N)
            def _():
                pltpu.sync_copy(msg_v, acc_sh.at[pl.ds(row0 + b * WB, WB)])
        plsc.subcore_barrier()

        g0 = wid * T

        def load(g, s):
            pltpu.sync_copy(ei_hbm.at[g], ei_v[s])
            pltpu.sync_copy(w_hbm.at[g], w_v[s])
            pltpu.async_copy(p_hbm.at[ei_v[s].at[0]], rows_v[s], sem[s])

        def wait_gather(s):
            pltpu.make_async_copy(
                p_hbm.at[ei_v[s].at[0]], rows_v[s], sem[s]).wait()

        def compute_scatter(s):
            rv = rows_v[s]
            wv = w_v[s]

            def _edges(j, carry2):
                wvec = [
                    plsc.load_gather(
                        wv, [jnp.full((L,), j + c * B, jnp.int32)])
                    for c in range(GK)
                ]
                for dd in range(D // L):
                    m = wvec[0] * rv[j, pl.ds(dd * L, L)]
                    for c in range(1, GK):
                        m = m + wvec[c] * rv[j, pl.ds(c * D + dd * L, L)]
                    msg_v[j, pl.ds(dd * L, L)] = m
                return carry2
            lax.fori_loop(0, B, _edges, 0)

            pltpu.sync_copy(msg_v, acc_sh.at[ei_v[s].at[1]], add=True)

        load(g0, 0)

        def pair_body(p, carry):
            g = g0 + 2 * p
            load(g + 1, 1)
            wait_gather(0)
            compute_scatter(0)

            @pl.when(p + 1 < T // 2)
            def _():
                load(g + 2, 0)
            wait_gather(1)
            compute_scatter(1)
            return carry
        lax.fori_loop(0, T // 2, pair_body, 0)

        plsc.subcore_barrier()
        for b in range(NBLK):
            @pl.when(row0 + b * WB < N)
            def _():
                r0 = row0 + b * WB
                pltpu.sync_copy(acc_sh.at[pl.ds(r0, WB)], msg_v)
                pltpu.sync_copy(msg_v, out_hbm.at[cid, pl.ds(r0, WB)])

    return sc_spmm, NC, B


def kernel(h, x, edge_weight, weights, inp_weights, graph_mixing_weight,
           inp_graph_mixing_weight, edge_index):
    N, D_IN = h.shape
    D_NET = x.shape[1]
    G, K, _, D_OUT = weights.shape
    E = edge_index.shape[1]
    GK = G * K
    DP = GK * D_OUT

    wh = weights * graph_mixing_weight[:, 0][:, None, None, None]
    wx = inp_weights * inp_graph_mixing_weight[:, 0][:, None, None, None]
    wh = wh.reshape(GK, D_IN, D_OUT).transpose(1, 0, 2).reshape(D_IN, DP)
    wx = wx.reshape(GK, D_NET, D_OUT).transpose(1, 0, 2).reshape(D_NET, DP)
    wcat = jnp.concatenate([wh, wx], axis=0)
    hx = jnp.concatenate([h, x], axis=1)

    BLK = 1000
    p = pl.pallas_call(
        _matmul_body,
        grid=(N // BLK,),
        in_specs=[
            pl.BlockSpec((BLK, D_IN + D_NET), lambda i: (i, 0)),
            pl.BlockSpec((D_IN + D_NET, DP), lambda i: (0, 0)),
        ],
        out_specs=pl.BlockSpec((BLK, DP), lambda i: (i, 0)),
        out_shape=jax.ShapeDtypeStruct((N, DP), jnp.float32),
    )(hx, wcat)

    sc_spmm, NC, B = _make_sc_spmm(N, E, GK, D_OUT, DP)
    TT = E // B
    eint = edge_index.reshape(2, TT, B).transpose(1, 0, 2)
    wpack = (edge_weight.reshape(GK, TT, B).transpose(1, 0, 2)
             .reshape(TT, GK * B))
    parts = sc_spmm(p, eint, wpack)

    out = pl.pallas_call(
        _addrelu_body,
        grid=(N // BLK,),
        in_specs=[
            pl.BlockSpec((BLK, D_OUT), lambda i: (i, 0)),
            pl.BlockSpec((BLK, D_OUT), lambda i: (i, 0)),
        ],
        out_specs=pl.BlockSpec((BLK, D_OUT), lambda i: (i, 0)),
        out_shape=jax.ShapeDtypeStruct((N, D_OUT), jnp.float32),
    )(parts[0], parts[1])
    return out


# speedup vs baseline: 10.7895x; 1.0751x over previous
import functools

import jax
import jax.numpy as jnp
from jax import lax
from jax.experimental import pallas as pl
from jax.experimental.pallas import tpu as pltpu
from jax.experimental.pallas import tpu_sc as plsc


def _matmul_body(hx_ref, w_ref, o_ref):
    o_ref[...] = jnp.dot(hx_ref[...], w_ref[...],
                         preferred_element_type=jnp.float32,
                         precision=jax.lax.Precision.HIGHEST
                         ).astype(jnp.bfloat16)


def _addrelu_body(a_ref, b_ref, o_ref):
    o_ref[...] = jnp.maximum(a_ref[...] + b_ref[...], 0.0)


def _make_sc_spmm(N, E, GK, D, DP):
    info = plsc.get_sparse_core_info()
    NC, NS, L = info.num_cores, info.num_subcores, info.num_lanes
    NW = NC * NS
    EW = E // NW
    B = 40
    T = EW // B
    WB = 40
    SB = ((N + NS - 1) // NS + WB - 1) // WB * WB
    NBLK = SB // WB
    assert E % NW == 0 and EW % B == 0 and N % WB == 0
    mesh = plsc.VectorSubcoreMesh(core_axis_name="c", subcore_axis_name="s")

    @functools.partial(
        pl.kernel,
        out_type=jax.ShapeDtypeStruct((NC, N, D), jnp.float32),
        mesh=mesh,
        compiler_params=pltpu.CompilerParams(needs_layout_passes=False),
        scratch_types=[
            pltpu.VMEM((2, B), jnp.int32),
            pltpu.VMEM((2, B), jnp.int32),
            pltpu.VMEM((GK * B,), jnp.float32),
            pltpu.VMEM((GK * B,), jnp.float32),
            pltpu.VMEM((B, DP // 2), jnp.int32),
            pltpu.VMEM((B, DP // 2), jnp.int32),
            pltpu.VMEM((B, D), jnp.float32),
            pltpu.VMEM_SHARED((N, D), jnp.float32),
            pltpu.SemaphoreType.DMA,
            pltpu.SemaphoreType.DMA,
        ],
    )
    def sc_spmm(p_hbm, ei_hbm, w_hbm, out_hbm,
                ei0_v, ei1_v, w0_v, w1_v, rows0_v, rows1_v, msg_v, acc_sh,
                sem0, sem1):
        cid = lax.axis_index("c")
        sid = lax.axis_index("s")
        wid = sid * NC + cid
        ei_v = (ei0_v, ei1_v)
        w_v = (w0_v, w1_v)
        rows_v = (rows0_v, rows1_v)
        sem = (sem0, sem1)

        def zero_row(r, carry):
            for dd in range(D // L):
                msg_v[r, pl.ds(dd * L, L)] = jnp.zeros((L,), jnp.float32)
            return carry
        lax.fori_loop(0, WB, zero_row, 0)
        row0 = sid * SB
        for b in range(NBLK):
            @pl.when(row0 + b * WB < N)
            def _():
                pltpu.sync_copy(msg_v, acc_sh.at[pl.ds(row0 + b * WB, WB)])
        plsc.subcore_barrier()

        g0 = wid * T

        def load(g, s):
            pltpu.sync_copy(ei_hbm.at[g], ei_v[s])
            pltpu.sync_copy(w_hbm.at[g], w_v[s])
            pltpu.async_copy(p_hbm.at[ei_v[s].at[0]], rows_v[s], sem[s])

        def wait_gather(s):
            pltpu.make_async_copy(
                p_hbm.at[ei_v[s].at[0]], rows_v[s], sem[s]).wait()

        def compute_scatter(s):
            rv = rows_v[s]
            wv = w_v[s]

            def _edges(j, carry2):
                wvec = [
                    plsc.load_gather(
                        wv, [jnp.full((L,), j + c * B, jnp.int32)])
                    for c in range(GK)
                ]
                for dd in range(D // (2 * L)):
                    ma = None
                    mb = None
                    for c in range(GK):
                        words = rv[j, pl.ds(c * (D // 2) + dd * L, L)]
                        ab = plsc.bitcast(words, jnp.bfloat16)
                        a, b = plsc.unpack(
                            ab, format=plsc.PackFormat.INTERLEAVED,
                            preferred_element_type=jnp.float32)
                        ma = wvec[c] * a if ma is None else ma + wvec[c] * a
                        mb = wvec[c] * b if mb is None else mb + wvec[c] * b
                    msg_v[j, pl.ds(dd * 2 * L, L)] = ma
                    msg_v[j, pl.ds(dd * 2 * L + L, L)] = mb
                return carry2
            lax.fori_loop(0, B, _edges, 0)

            pltpu.sync_copy(msg_v, acc_sh.at[ei_v[s].at[1]], add=True)

        load(g0, 0)

        def pair_body(p, carry):
            g = g0 + 2 * p
            load(g + 1, 1)
            wait_gather(0)
            compute_scatter(0)

            @pl.when(2 * p + 2 < T)
            def _():
                load(g + 2, 0)
            wait_gather(1)
            compute_scatter(1)
            return carry
        lax.fori_loop(0, T // 2, pair_body, 0)
        if T % 2 == 1:
            wait_gather(0)
            compute_scatter(0)

        plsc.subcore_barrier()
        for b in range(NBLK):
            @pl.when(row0 + b * WB < N)
            def _():
                r0 = row0 + b * WB
                pltpu.sync_copy(acc_sh.at[pl.ds(r0, WB)], msg_v)
                pltpu.sync_copy(msg_v, out_hbm.at[cid, pl.ds(r0, WB)])

    return sc_spmm, NC, B


def kernel(h, x, edge_weight, weights, inp_weights, graph_mixing_weight,
           inp_graph_mixing_weight, edge_index):
    N, D_IN = h.shape
    D_NET = x.shape[1]
    G, K, _, D_OUT = weights.shape
    E = edge_index.shape[1]
    GK = G * K
    DP = GK * D_OUT

    wh = weights * graph_mixing_weight[:, 0][:, None, None, None]
    wx = inp_weights * inp_graph_mixing_weight[:, 0][:, None, None, None]
    wh = wh.reshape(GK, D_IN, D_OUT).transpose(1, 0, 2).reshape(D_IN, DP)
    wx = wx.reshape(GK, D_NET, D_OUT).transpose(1, 0, 2).reshape(D_NET, DP)
    wcat = jnp.concatenate([wh, wx], axis=0)
    m = jnp.arange(D_OUT)
    jcol = 16 * (2 * (m // 32) + (m % 32) % 2) + (m % 32) // 2
    colperm = jnp.concatenate([c * D_OUT + jcol for c in range(GK)])
    wcat = wcat[:, colperm]
    hx = jnp.concatenate([h, x], axis=1)

    BLK = 1000
    p = pl.pallas_call(
        _matmul_body,
        grid=(N // BLK,),
        in_specs=[
            pl.BlockSpec((BLK, D_IN + D_NET), lambda i: (i, 0)),
            pl.BlockSpec((D_IN + D_NET, DP), lambda i: (0, 0)),
        ],
        out_specs=pl.BlockSpec((BLK, DP), lambda i: (i, 0)),
        out_shape=jax.ShapeDtypeStruct((N, DP), jnp.bfloat16),
    )(hx, wcat)

    sc_spmm, NC, B = _make_sc_spmm(N, E, GK, D_OUT, DP)
    TT = E // B
    eint = edge_index.reshape(2, TT, B).transpose(1, 0, 2)
    wpack = (edge_weight.reshape(GK, TT, B).transpose(1, 0, 2)
             .reshape(TT, GK * B))
    p32 = jax.lax.bitcast_convert_type(
        p.reshape(N, DP // 2, 2), jnp.int32)
    parts = sc_spmm(p32, eint, wpack)

    out = pl.pallas_call(
        _addrelu_body,
        grid=(N // BLK,),
        in_specs=[
            pl.BlockSpec((BLK, D_OUT), lambda i: (i, 0)),
            pl.BlockSpec((BLK, D_OUT), lambda i: (i, 0)),
        ],
        out_specs=pl.BlockSpec((BLK, D_OUT), lambda i: (i, 0)),
        out_shape=jax.ShapeDtypeStruct((N, D_OUT), jnp.float32),
    )(parts[0], parts[1])
    return out


# speedup vs baseline: 13.0979x; 1.2140x over previous
import functools

import jax
import jax.numpy as jnp
from jax import lax
from jax.experimental import pallas as pl
from jax.experimental.pallas import tpu as pltpu
from jax.experimental.pallas import tpu_sc as plsc


def _matmul_body(hx_ref, w_ref, o_ref):
    o_ref[...] = jnp.dot(hx_ref[...], w_ref[...],
                         preferred_element_type=jnp.float32,
                         precision=jax.lax.Precision.HIGHEST
                         ).astype(jnp.bfloat16)


def _addrelu_body(a_ref, b_ref, o_ref):
    o_ref[...] = jnp.maximum(a_ref[...] + b_ref[...], 0.0)


def _make_sc_spmm(N, E, GK, D, DP):
    info = plsc.get_sparse_core_info()
    NC, NS, L = info.num_cores, info.num_subcores, info.num_lanes
    NW = NC * NS
    B = 64
    TT = E // B
    TBASE = TT // NW
    TREM = TT % NW
    WB = 40
    SB = ((N + NS - 1) // NS + WB - 1) // WB * WB
    NBLK = SB // WB
    assert E % B == 0 and N % WB == 0
    mesh = plsc.VectorSubcoreMesh(core_axis_name="c", subcore_axis_name="s")

    @functools.partial(
        pl.kernel,
        out_type=jax.ShapeDtypeStruct((NC, N, D), jnp.float32),
        mesh=mesh,
        compiler_params=pltpu.CompilerParams(needs_layout_passes=False),
        scratch_types=[
            pltpu.VMEM((2 + GK, B), jnp.int32),
            pltpu.VMEM((2 + GK, B), jnp.int32),
            pltpu.VMEM((B, DP // 2), jnp.int32),
            pltpu.VMEM((B, DP // 2), jnp.int32),
            pltpu.VMEM((B, D), jnp.float32),
            pltpu.VMEM_SHARED((N, D), jnp.float32),
            pltpu.SemaphoreType.DMA,
            pltpu.SemaphoreType.DMA,
        ],
    )
    def sc_spmm(p_hbm, rec_hbm, out_hbm,
                rec0_v, rec1_v, rows0_v, rows1_v, msg_v, acc_sh,
                sem0, sem1):
        cid = lax.axis_index("c")
        sid = lax.axis_index("s")
        wid = sid * NC + cid
        rec_v = (rec0_v, rec1_v)
        rows_v = (rows0_v, rows1_v)
        sem = (sem0, sem1)

        def zero_row(r, carry):
            for dd in range(D // L):
                msg_v[r, pl.ds(dd * L, L)] = jnp.zeros((L,), jnp.float32)
            return carry
        lax.fori_loop(0, WB, zero_row, 0)
        row0 = sid * SB
        for b in range(NBLK):
            @pl.when(row0 + b * WB < N)
            def _():
                pltpu.sync_copy(msg_v.at[pl.ds(0, WB)],
                                acc_sh.at[pl.ds(row0 + b * WB, WB)])
        plsc.subcore_barrier()

        g0 = TBASE * wid + jnp.minimum(wid, TREM)
        tcnt = TBASE + (wid < TREM).astype(jnp.int32)

        def load(g, s):
            pltpu.sync_copy(rec_hbm.at[g], rec_v[s])
            pltpu.async_copy(p_hbm.at[rec_v[s].at[0]], rows_v[s], sem[s])

        def wait_gather(s):
            pltpu.make_async_copy(
                p_hbm.at[rec_v[s].at[0]], rows_v[s], sem[s]).wait()

        def compute_scatter(s):
            rv = rows_v[s]
            wv = rec_v[s]

            def _edges(j, carry2):
                wvec = [
                    plsc.bitcast(
                        plsc.load_gather(
                            wv, [jnp.full((L,), 2 + c, jnp.int32),
                                 jnp.full((L,), j, jnp.int32)]),
                        jnp.float32)
                    for c in range(GK)
                ]
                for dd in range(D // (2 * L)):
                    ma = None
                    mb = None
                    for c in range(GK):
                        words = rv[j, pl.ds(c * (D // 2) + dd * L, L)]
                        ab = plsc.bitcast(words, jnp.bfloat16)
                        a, b = plsc.unpack(
                            ab, format=plsc.PackFormat.INTERLEAVED,
                            preferred_element_type=jnp.float32)
                        ma = wvec[c] * a if ma is None else ma + wvec[c] * a
                        mb = wvec[c] * b if mb is None else mb + wvec[c] * b
                    msg_v[j, pl.ds(dd * 2 * L, L)] = ma
                    msg_v[j, pl.ds(dd * 2 * L + L, L)] = mb
                return carry2
            lax.fori_loop(0, B, _edges, 0)

            pltpu.sync_copy(msg_v, acc_sh.at[rec_v[s].at[1]], add=True)

        load(g0, 0)

        def pair_body(p, carry):
            g = g0 + 2 * p
            load(g + 1, 1)
            wait_gather(0)
            compute_scatter(0)

            @pl.when(2 * p + 2 < tcnt)
            def _():
                load(g + 2, 0)
            wait_gather(1)
            compute_scatter(1)
            return carry
        lax.fori_loop(0, tcnt // 2, pair_body, 0)

        @pl.when(tcnt % 2 == 1)
        def _():
            wait_gather(0)
            compute_scatter(0)

        plsc.subcore_barrier()
        for b in range(NBLK):
            @pl.when(row0 + b * WB < N)
            def _():
                r0 = row0 + b * WB
                pltpu.sync_copy(acc_sh.at[pl.ds(r0, WB)],
                                msg_v.at[pl.ds(0, WB)])
                pltpu.sync_copy(msg_v.at[pl.ds(0, WB)],
                                out_hbm.at[cid, pl.ds(r0, WB)])

    return sc_spmm, NC, B


def kernel(h, x, edge_weight, weights, inp_weights, graph_mixing_weight,
           inp_graph_mixing_weight, edge_index):
    N, D_IN = h.shape
    D_NET = x.shape[1]
    G, K, _, D_OUT = weights.shape
    E = edge_index.shape[1]
    GK = G * K
    DP = GK * D_OUT

    wh = weights * graph_mixing_weight[:, 0][:, None, None, None]
    wx = inp_weights * inp_graph_mixing_weight[:, 0][:, None, None, None]
    wh = wh.reshape(GK, D_IN, D_OUT).transpose(1, 0, 2).reshape(D_IN, DP)
    wx = wx.reshape(GK, D_NET, D_OUT).transpose(1, 0, 2).reshape(D_NET, DP)
    wcat = jnp.concatenate([wh, wx], axis=0)
    m = jnp.arange(D_OUT)
    jcol = 16 * (2 * (m // 32) + (m % 32) % 2) + (m % 32) // 2
    colperm = jnp.concatenate([c * D_OUT + jcol for c in range(GK)])
    wcat = wcat[:, colperm]
    hx = jnp.concatenate([h, x], axis=1)

    BLK = 1000
    p = pl.pallas_call(
        _matmul_body,
        grid=(N // BLK,),
        in_specs=[
            pl.BlockSpec((BLK, D_IN + D_NET), lambda i: (i, 0)),
            pl.BlockSpec((D_IN + D_NET, DP), lambda i: (0, 0)),
        ],
        out_specs=pl.BlockSpec((BLK, DP), lambda i: (i, 0)),
        out_shape=jax.ShapeDtypeStruct((N, DP), jnp.bfloat16),
    )(hx, wcat)

    sc_spmm, NC, B = _make_sc_spmm(N, E, GK, D_OUT, DP)
    TT = E // B
    eint = edge_index.reshape(2, TT, B).transpose(1, 0, 2)
    wbits = jax.lax.bitcast_convert_type(
        edge_weight.reshape(GK, TT, B), jnp.int32).transpose(1, 0, 2)
    rec = jnp.concatenate([eint, wbits], axis=1)
    p32 = jax.lax.bitcast_convert_type(
        p.reshape(N, DP // 2, 2), jnp.int32)
    parts = sc_spmm(p32, rec)

    out = pl.pallas_call(
        _addrelu_body,
        grid=(N // BLK,),
        in_specs=[
            pl.BlockSpec((BLK, D_OUT), lambda i: (i, 0)),
            pl.BlockSpec((BLK, D_OUT), lambda i: (i, 0)),
        ],
        out_specs=pl.BlockSpec((BLK, D_OUT), lambda i: (i, 0)),
        out_shape=jax.ShapeDtypeStruct((N, D_OUT), jnp.float32),
    )(parts[0], parts[1])
    return out
